# baseline (speedup 1.0000x reference)
import functools

import jax
import jax.numpy as jnp
from jax import lax
from jax.experimental import pallas as pl
from jax.experimental.pallas import tpu as pltpu
from jax.experimental.pallas import tpu_sc as plsc

N = 16384
D_ATOM = 16
D_CHARGE = 32
D_NUM = 16
D_OUT = D_ATOM + D_CHARGE + D_NUM
CH = 128


def _build(nc, ns):
    nw = nc * ns
    bpw = N // nw
    nlin = bpw // CH
    xlin = bpw * D_NUM // 128
    olin = bpw // 2
    mesh = plsc.VectorSubcoreMesh(core_axis_name="c", subcore_axis_name="s")

    @functools.partial(
        pl.kernel,
        mesh=mesh,
        out_type=jax.ShapeDtypeStruct((N // 2, 128), jnp.float32),
        scratch_types=[
            pltpu.VMEM((nlin, CH), jnp.int32),
            pltpu.VMEM((nlin, CH), jnp.int32),
            pltpu.VMEM((nlin, CH), jnp.int32),
            pltpu.VMEM((nlin, CH), jnp.int32),
            pltpu.VMEM((xlin, 128), jnp.float32),
            pltpu.VMEM((CH, 128), jnp.float32),
            pltpu.VMEM((CH, 128), jnp.float32),
            pltpu.VMEM((olin, 128), jnp.float32),
            pltpu.SemaphoreType.DMA,
        ],
    )
    def k(wa_hbm, wc_hbm, iab_hbm, icb_hbm, offa_hbm, offc_hbm, ex_hbm,
          out_hbm, iab_v, icb_v, offa_v, offc_v, ex_v, la_v, lc_v, out_v,
          sem):
        wid = lax.axis_index("s") * nc + lax.axis_index("c")
        pltpu.sync_copy(iab_hbm.at[wid], iab_v)
        pltpu.sync_copy(icb_hbm.at[wid], icb_v)
        pltpu.sync_copy(offa_hbm.at[wid], offa_v)
        pltpu.sync_copy(offc_hbm.at[wid], offc_v)
        pltpu.sync_copy(ex_hbm.at[wid], ex_v)

        def chunk(j, carry):
            cpa = pltpu.async_copy(wa_hbm.at[iab_v.at[j]], la_v, sem)
            cpc = pltpu.async_copy(wc_hbm.at[icb_v.at[j]], lc_v, sem)
            cpa.wait()
            cpc.wait()
            for g in range(CH // 16):
                c0 = g * 16
                oa16 = offa_v[j, pl.ds(c0, 16)] * D_ATOM
                oc16 = offc_v[j, pl.ds(c0, 16)] * D_CHARGE
                for i in range(16):
                    lr = c0 + i
                    b0 = (lr & 1) * 64
                    sa = oa16[i]
                    sc = oc16[i]
                    ol = j * (CH // 2) + (lr >> 1)
                    out_v[ol, pl.ds(b0, D_ATOM)] = la_v[lr, pl.ds(sa, 16)]
                    out_v[ol, pl.ds(b0 + 16, 16)] = lc_v[lr, pl.ds(sc, 16)]
                    out_v[ol, pl.ds(b0 + 32, 16)] = (
                        lc_v[lr, pl.ds(sc + 16, 16)]
                    )
                    out_v[ol, pl.ds(b0 + 48, D_NUM)] = (
                        ex_v[j * (CH // 8) + (lr >> 3),
                             pl.ds((lr & 7) * D_NUM, D_NUM)]
                    )
            return carry

        lax.fori_loop(0, nlin, chunk, 0)
        pltpu.sync_copy(out_v, out_hbm.at[pl.ds(wid * olin, olin)])

    return k, nw, nlin, xlin


def kernel(pos, extra_feat, W_atom, W_charge, atom_type, charge_state):
    info = plsc.get_sparse_core_info()
    k, nw, nlin, xlin = _build(info.num_cores, info.num_subcores)
    wa2 = W_atom.reshape(-1, 128)
    wc2 = W_charge.reshape(-1, 128)
    iab = (atom_type >> 3).reshape(nw, nlin, CH)
    icb = (charge_state >> 2).reshape(nw, nlin, CH)
    offa = (atom_type & 7).reshape(nw, nlin, CH)
    offc = (charge_state & 3).reshape(nw, nlin, CH)
    ex = extra_feat.reshape(nw, xlin, 128)
    out = k(wa2, wc2, iab, icb, offa, offc, ex)
    return out.reshape(N, D_OUT).astype(pos.dtype)


# speedup vs baseline: 1.0024x; 1.0024x over previous
import functools

import jax
import jax.numpy as jnp
from jax import lax
from jax.experimental import pallas as pl
from jax.experimental.pallas import tpu as pltpu
from jax.experimental.pallas import tpu_sc as plsc

N = 16384
D_ATOM = 16
D_CHARGE = 32
D_NUM = 16
D_OUT = D_ATOM + D_CHARGE + D_NUM
CH = 128


def _build(nc, ns):
    nw = nc * ns
    bpw = N // nw
    nlin = bpw // CH
    xlin = bpw * D_NUM // 128
    olin = bpw // 2
    mesh = plsc.VectorSubcoreMesh(core_axis_name="c", subcore_axis_name="s")

    @functools.partial(
        pl.kernel,
        mesh=mesh,
        out_type=jax.ShapeDtypeStruct((N // 2, 128), jnp.float32),
        scratch_types=[
            pltpu.VMEM((4 * nlin, CH), jnp.int32),
            pltpu.VMEM((xlin, 128), jnp.float32),
            pltpu.VMEM((CH, 128), jnp.float32),
            pltpu.VMEM((CH, 128), jnp.float32),
            pltpu.VMEM((olin, 128), jnp.float32),
            pltpu.SemaphoreType.DMA,
        ],
        compiler_params=pltpu.CompilerParams(use_tc_tiling_on_sc=True),
    )
    def k(wa_hbm, wc_hbm, idx_hbm, ex_hbm,
          out_hbm, idx_v, ex_v, la_v, lc_v, out_v,
          sem):
        wid = lax.axis_index("s") * nc + lax.axis_index("c")
        pltpu.sync_copy(idx_hbm.at[wid], idx_v)
        pltpu.sync_copy(ex_hbm.at[wid], ex_v)

        def chunk(j, carry):
            cpa = pltpu.async_copy(wa_hbm.at[idx_v.at[j]], la_v, sem)
            cpc = pltpu.async_copy(wc_hbm.at[idx_v.at[nlin + j]], lc_v, sem)
            cpa.wait()
            cpc.wait()
            for g in range(CH // 16):
                c0 = g * 16
                oa16 = idx_v[2 * nlin + j, pl.ds(c0, 16)] * D_ATOM
                oc16 = idx_v[3 * nlin + j, pl.ds(c0, 16)] * D_CHARGE
                for i in range(16):
                    lr = c0 + i
                    b0 = (lr & 1) * 64
                    sa = oa16[i]
                    sc = oc16[i]
                    ol = j * (CH // 2) + (lr >> 1)
                    out_v[ol, pl.ds(b0, D_ATOM)] = la_v[lr, pl.ds(sa, 16)]
                    out_v[ol, pl.ds(b0 + 16, 16)] = lc_v[lr, pl.ds(sc, 16)]
                    out_v[ol, pl.ds(b0 + 32, 16)] = (
                        lc_v[lr, pl.ds(sc + 16, 16)]
                    )
                    out_v[ol, pl.ds(b0 + 48, D_NUM)] = (
                        ex_v[j * (CH // 8) + (lr >> 3),
                             pl.ds((lr & 7) * D_NUM, D_NUM)]
                    )
            return carry

        lax.fori_loop(0, nlin, chunk, 0)
        pltpu.sync_copy(out_v, out_hbm.at[pl.ds(wid * olin, olin)])

    return k, nw, nlin, xlin


def kernel(pos, extra_feat, W_atom, W_charge, atom_type, charge_state):
    info = plsc.get_sparse_core_info()
    k, nw, nlin, xlin = _build(info.num_cores, info.num_subcores)
    wa2 = W_atom.reshape(-1, 128)
    wc2 = W_charge.reshape(-1, 128)
    idx = jnp.concatenate(
        [
            (atom_type >> 3).reshape(nw, nlin, CH),
            (charge_state >> 2).reshape(nw, nlin, CH),
            (atom_type & 7).reshape(nw, nlin, CH),
            (charge_state & 3).reshape(nw, nlin, CH),
        ],
        axis=1,
    )
    ex = extra_feat.reshape(nw, xlin, 128)
    out = k(wa2, wc2, idx, ex)
    return out.reshape(N, D_OUT).astype(pos.dtype)


# speedup vs baseline: 1.4973x; 1.4936x over previous
import functools

import jax
import jax.numpy as jnp
from jax import lax
from jax.experimental import pallas as pl
from jax.experimental.pallas import tpu as pltpu
from jax.experimental.pallas import tpu_sc as plsc

N = 16384
D_ATOM = 16
D_CHARGE = 32
D_NUM = 16
D_OUT = D_ATOM + D_CHARGE + D_NUM
CH = 128


def _build(nc, ns):
    nw = nc * ns
    bpw = N // nw
    nlin = bpw // CH
    olin = bpw // 2
    mesh = plsc.VectorSubcoreMesh(core_axis_name="c", subcore_axis_name="s")

    @functools.partial(
        pl.kernel,
        mesh=mesh,
        out_type=jax.ShapeDtypeStruct((N // 2, 128), jnp.float32),
        scratch_types=[
            pltpu.VMEM((2 * nlin, CH), jnp.int32),
            pltpu.VMEM((olin, 128), jnp.float32),
            pltpu.SemaphoreType.DMA,
        ],
        compiler_params=pltpu.CompilerParams(use_tc_tiling_on_sc=True),
    )
    def k(wa_hbm, wc_hbm, idx_hbm, ex_hbm, out_hbm, idx_v, out_v, sem):
        wid = lax.axis_index("s") * nc + lax.axis_index("c")
        pltpu.sync_copy(idx_hbm.at[wid], idx_v)
        row0 = wid * bpw

        def chunk(j, carry):
            cps = []
            for g in range(CH // 16):
                c0 = g * 16
                ia16 = idx_v[j, pl.ds(c0, 16)]
                ic16 = idx_v[nlin + j, pl.ds(c0, 16)]
                for i in range(16):
                    lr = c0 + i
                    b0 = (lr & 1) * 64
                    ol = j * (CH // 2) + (lr >> 1)
                    er = row0 + j * CH + lr
                    cps.append(pltpu.async_copy(
                        wa_hbm.at[ia16[i]],
                        out_v.at[ol, pl.ds(b0, D_ATOM)], sem))
                    cps.append(pltpu.async_copy(
                        wc_hbm.at[ic16[i]],
                        out_v.at[ol, pl.ds(b0 + 16, D_CHARGE)], sem))
                    cps.append(pltpu.async_copy(
                        ex_hbm.at[er],
                        out_v.at[ol, pl.ds(b0 + 48, D_NUM)], sem))
            for cp in cps:
                cp.wait()
            return carry

        lax.fori_loop(0, nlin, chunk, 0)
        pltpu.sync_copy(out_v, out_hbm.at[pl.ds(wid * olin, olin)])

    return k, nw, nlin


def kernel(pos, extra_feat, W_atom, W_charge, atom_type, charge_state):
    info = plsc.get_sparse_core_info()
    k, nw, nlin = _build(info.num_cores, info.num_subcores)
    idx = jnp.concatenate(
        [
            atom_type.reshape(nw, nlin, CH),
            charge_state.reshape(nw, nlin, CH),
        ],
        axis=1,
    )
    out = k(W_atom, W_charge, idx, extra_feat)
    return out.reshape(N, D_OUT).astype(pos.dtype)


# speedup vs baseline: 1.5415x; 1.0295x over previous
import functools

import jax
import jax.numpy as jnp
from jax import lax
from jax.experimental import pallas as pl
from jax.experimental.pallas import tpu as pltpu
from jax.experimental.pallas import tpu_sc as plsc

N = 16384
D_ATOM = 16
D_CHARGE = 32
D_NUM = 16
D_OUT = D_ATOM + D_CHARGE + D_NUM
CH = 128


def _build(nc, ns):
    nw = nc * ns
    bpw = N // nw
    nlin = bpw // CH
    olin = bpw // 2
    mesh = plsc.VectorSubcoreMesh(core_axis_name="c", subcore_axis_name="s")

    @functools.partial(
        pl.kernel,
        mesh=mesh,
        out_type=jax.ShapeDtypeStruct((N // 2, 128), jnp.float32),
        scratch_types=[
            pltpu.VMEM((2 * nlin, CH), jnp.int32),
            pltpu.VMEM((olin, 128), jnp.float32),
            pltpu.SemaphoreType.DMA,
        ],
        compiler_params=pltpu.CompilerParams(use_tc_tiling_on_sc=True),
    )
    def k(wa_hbm, wc_hbm, idx_hbm, ex_hbm, out_hbm, idx_v, out_v, sem):
        wid = lax.axis_index("s") * nc + lax.axis_index("c")
        pltpu.sync_copy(idx_hbm.at[wid], idx_v)
        row0 = wid * bpw

        def chunk(j, carry):
            for g in range(CH // 16):
                c0 = g * 16
                ia16 = idx_v[j, pl.ds(c0, 16)]
                ic16 = idx_v[nlin + j, pl.ds(c0, 16)]
                for i in range(16):
                    lr = c0 + i
                    b0 = (lr & 1) * 64
                    ol = j * (CH // 2) + (lr >> 1)
                    er = row0 + j * CH + lr
                    pltpu.async_copy(
                        wa_hbm.at[ia16[i]],
                        out_v.at[ol, pl.ds(b0, D_ATOM)], sem)
                    pltpu.async_copy(
                        wc_hbm.at[ic16[i]],
                        out_v.at[ol, pl.ds(b0 + 16, D_CHARGE)], sem)
                    pltpu.async_copy(
                        ex_hbm.at[er],
                        out_v.at[ol, pl.ds(b0 + 48, D_NUM)], sem)
            return carry

        lax.fori_loop(0, nlin, chunk, 0)
        pltpu.make_async_copy(
            out_hbm.at[pl.ds(wid * olin, olin)], out_v, sem).wait()
        pltpu.sync_copy(out_v, out_hbm.at[pl.ds(wid * olin, olin)])

    return k, nw, nlin


def kernel(pos, extra_feat, W_atom, W_charge, atom_type, charge_state):
    info = plsc.get_sparse_core_info()
    k, nw, nlin = _build(info.num_cores, info.num_subcores)
    idx = jnp.concatenate(
        [
            atom_type.reshape(nw, nlin, CH),
            charge_state.reshape(nw, nlin, CH),
        ],
        axis=1,
    )
    out = k(W_atom, W_charge, idx, extra_feat)
    return out.reshape(N, D_OUT).astype(pos.dtype)
